# baseline (device time: 51503 ns/iter reference)
import functools

import jax
import jax.numpy as jnp
from jax import lax
from jax.experimental import pallas as pl
from jax.experimental.pallas import tpu as pltpu

N_DEV = 8
XOR_MASKS = (1, 3, 4)
B, SQ, D_MODEL = 2, 256, 512
H_LOCAL, DH = 4, 64
D_HEADS = H_LOCAL * DH


def kernel(x, Wq, K_ext, V_ext, Wo):
    my = lax.axis_index("i")
    wq = lax.dynamic_slice_in_dim(Wq, my * D_HEADS, D_HEADS, axis=1)
    wo = lax.dynamic_slice_in_dim(Wo, my * D_HEADS, D_HEADS, axis=0)
    x2 = x.reshape(B * SQ, D_MODEL).astype(jnp.bfloat16)
    wq = wq.astype(jnp.bfloat16)
    wo = wo.astype(jnp.bfloat16)
    k = jnp.swapaxes(K_ext, 1, 2).reshape(B * H_LOCAL, SQ, DH).astype(jnp.bfloat16)
    v = jnp.swapaxes(V_ext, 1, 2).reshape(B * H_LOCAL, SQ, DH).astype(jnp.bfloat16)

    out = pl.pallas_call(
        _body,
        out_shape=jax.ShapeDtypeStruct((B * SQ, D_MODEL), jnp.float32),
        in_specs=[pl.BlockSpec(memory_space=pltpu.VMEM)] * 5,
        out_specs=pl.BlockSpec(memory_space=pltpu.VMEM),
        scratch_shapes=[
            pltpu.VMEM((B * SQ, D_HEADS), jnp.bfloat16),
            pltpu.VMEM((3, B * SQ, D_MODEL), jnp.float32),
            pltpu.SemaphoreType.DMA((3,)),
            pltpu.SemaphoreType.DMA((3,)),
        ],
        compiler_params=pltpu.CompilerParams(collective_id=0),
    )(x2, wq, k, v, wo)
    return out.reshape(B, SQ, D_MODEL)


def _body(x_ref, wq_ref, k_ref, v_ref, wo_ref, out_ref, ctx_ref, recv_ref,
          send_sems, recv_sems):
    my = lax.axis_index("i")

    barrier = pltpu.get_barrier_semaphore()
    for m in XOR_MASKS:
        pl.semaphore_signal(barrier, inc=1, device_id=(my ^ m,),
                            device_id_type=pl.DeviceIdType.MESH)
    pl.semaphore_wait(barrier, len(XOR_MASKS))

    qb = lax.broadcasted_iota(jnp.int32, (SQ, SQ), 0) // 64
    kb = lax.broadcasted_iota(jnp.int32, (SQ, SQ), 1) // 64
    mask = (qb == kb) | (kb == 0) | ((qb + kb) % 3 == 0)

    for b in range(B):
        xb = x_ref[b * SQ:(b + 1) * SQ, :]
        q_b = jnp.dot(xb, wq_ref[...], preferred_element_type=jnp.float32)
        q_b = q_b.astype(jnp.bfloat16)
        for h in range(H_LOCAL):
            qh = q_b[:, h * DH:(h + 1) * DH]
            kh = k_ref[b * H_LOCAL + h]
            s = lax.dot_general(qh, kh, (((1,), (1,)), ((), ())),
                                preferred_element_type=jnp.float32) * 0.125
            s = jnp.where(mask, s, -1e9)
            e = jnp.exp(s - jnp.max(s, axis=1, keepdims=True))
            w = (e / jnp.sum(e, axis=1, keepdims=True)).astype(jnp.bfloat16)
            ctx = jnp.dot(w, v_ref[b * H_LOCAL + h],
                          preferred_element_type=jnp.float32)
            ctx_ref[b * SQ:(b + 1) * SQ, h * DH:(h + 1) * DH] = (
                ctx.astype(jnp.bfloat16))

    out_ref[...] = jnp.dot(ctx_ref[...], wo_ref[...],
                           preferred_element_type=jnp.float32)

    for r, m in enumerate(XOR_MASKS):
        rdma = pltpu.make_async_remote_copy(
            src_ref=out_ref,
            dst_ref=recv_ref.at[r],
            send_sem=send_sems.at[r],
            recv_sem=recv_sems.at[r],
            device_id=(my ^ m,),
            device_id_type=pl.DeviceIdType.MESH,
        )
        rdma.start()
        rdma.wait()
        out_ref[...] = out_ref[...] + recv_ref[r]

    @functools.partial(pl.run_scoped, exit_sem=pltpu.SemaphoreType.REGULAR)
    def _(exit_sem):
        for m in XOR_MASKS:
            pl.semaphore_signal(exit_sem, inc=1, device_id=(my ^ m,),
                                device_id_type=pl.DeviceIdType.MESH)
        pl.semaphore_wait(exit_sem, len(XOR_MASKS))


# device time: 34691 ns/iter; 1.4846x vs baseline; 1.4846x over previous
import functools

import jax
import jax.numpy as jnp
from jax import lax
from jax.experimental import pallas as pl
from jax.experimental.pallas import tpu as pltpu

N_DEV = 8
XOR_MASKS = (1, 3, 4)
B, SQ, D_MODEL = 2, 256, 512
H_LOCAL, DH = 4, 64
D_HEADS = H_LOCAL * DH


def kernel(x, Wq, K_ext, V_ext, Wo):
    my = lax.axis_index("i")
    wq = lax.dynamic_slice_in_dim(Wq, my * D_HEADS, D_HEADS, axis=1)
    wo = lax.dynamic_slice_in_dim(Wo, my * D_HEADS, D_HEADS, axis=0)
    x2 = x.reshape(B * SQ, D_MODEL).astype(jnp.bfloat16)
    wq = wq.astype(jnp.bfloat16)
    wo = wo.astype(jnp.bfloat16)
    k = jnp.swapaxes(K_ext, 1, 2).reshape(B * H_LOCAL, SQ, DH).astype(jnp.bfloat16)
    v = jnp.swapaxes(V_ext, 1, 2).reshape(B * H_LOCAL, SQ, DH).astype(jnp.bfloat16)

    out = pl.pallas_call(
        _body,
        out_shape=jax.ShapeDtypeStruct((B * SQ, D_MODEL), jnp.float32),
        in_specs=[pl.BlockSpec(memory_space=pltpu.VMEM)] * 5,
        out_specs=pl.BlockSpec(memory_space=pltpu.VMEM),
        scratch_shapes=[
            pltpu.VMEM((B * SQ, D_HEADS), jnp.bfloat16),
            pltpu.VMEM((B * SQ, D_MODEL), jnp.bfloat16),
            pltpu.VMEM((3, B * SQ, D_MODEL), jnp.bfloat16),
            pltpu.SemaphoreType.DMA((3,)),
            pltpu.SemaphoreType.DMA((3,)),
        ],
        compiler_params=pltpu.CompilerParams(collective_id=0),
    )(x2, wq, k, v, wo)
    return out.reshape(B, SQ, D_MODEL)


def _body(x_ref, wq_ref, k_ref, v_ref, wo_ref, out_ref, ctx_ref, send_ref,
          recv_ref, send_sems, recv_sems):
    my = lax.axis_index("i")

    barrier = pltpu.get_barrier_semaphore()
    for m in XOR_MASKS:
        pl.semaphore_signal(barrier, inc=1, device_id=(my ^ m,),
                            device_id_type=pl.DeviceIdType.MESH)
    pl.semaphore_wait(barrier, len(XOR_MASKS))

    qb = lax.broadcasted_iota(jnp.int32, (SQ, SQ), 0) // 64
    kb = lax.broadcasted_iota(jnp.int32, (SQ, SQ), 1) // 64
    mask = (qb == kb) | (kb == 0) | ((qb + kb) % 3 == 0)

    for b in range(B):
        xb = x_ref[b * SQ:(b + 1) * SQ, :]
        q_b = jnp.dot(xb, wq_ref[...], preferred_element_type=jnp.float32)
        q_b = q_b.astype(jnp.bfloat16)
        for h in range(H_LOCAL):
            qh = q_b[:, h * DH:(h + 1) * DH]
            kh = k_ref[b * H_LOCAL + h]
            s = lax.dot_general(qh, kh, (((1,), (1,)), ((), ())),
                                preferred_element_type=jnp.float32) * 0.125
            s = jnp.where(mask, s, -1e9)
            e = jnp.exp(s - jnp.max(s, axis=1, keepdims=True))
            w = (e / jnp.sum(e, axis=1, keepdims=True)).astype(jnp.bfloat16)
            ctx = jnp.dot(w, v_ref[b * H_LOCAL + h],
                          preferred_element_type=jnp.float32)
            ctx_ref[b * SQ:(b + 1) * SQ, h * DH:(h + 1) * DH] = (
                ctx.astype(jnp.bfloat16))

    out_ref[...] = jnp.dot(ctx_ref[...], wo_ref[...],
                           preferred_element_type=jnp.float32)

    for r, m in enumerate(XOR_MASKS):
        send_ref[...] = out_ref[...].astype(jnp.bfloat16)
        rdma = pltpu.make_async_remote_copy(
            src_ref=send_ref,
            dst_ref=recv_ref.at[r],
            send_sem=send_sems.at[r],
            recv_sem=recv_sems.at[r],
            device_id=(my ^ m,),
            device_id_type=pl.DeviceIdType.MESH,
        )
        rdma.start()
        rdma.wait()
        out_ref[...] = out_ref[...] + recv_ref[r].astype(jnp.float32)

    @functools.partial(pl.run_scoped, exit_sem=pltpu.SemaphoreType.REGULAR)
    def _(exit_sem):
        for m in XOR_MASKS:
            pl.semaphore_signal(exit_sem, inc=1, device_id=(my ^ m,),
                                device_id_type=pl.DeviceIdType.MESH)
        pl.semaphore_wait(exit_sem, len(XOR_MASKS))


# device time: 26896 ns/iter; 1.9149x vs baseline; 1.2898x over previous
import functools

import jax
import jax.numpy as jnp
from jax import lax
from jax.experimental import pallas as pl
from jax.experimental.pallas import tpu as pltpu

N_DEV = 8
XOR_MASKS = (1, 3, 4)
N_CHUNKS = 4
CHUNK = 512 // N_CHUNKS
B, SQ, D_MODEL = 2, 256, 512
H_LOCAL, DH = 4, 64
D_HEADS = H_LOCAL * DH


def kernel(x, Wq, K_ext, V_ext, Wo):
    my = lax.axis_index("i")
    wq = lax.dynamic_slice_in_dim(Wq, my * D_HEADS, D_HEADS, axis=1)
    wo = lax.dynamic_slice_in_dim(Wo, my * D_HEADS, D_HEADS, axis=0)
    x2 = x.reshape(B * SQ, D_MODEL).astype(jnp.bfloat16)
    wq = wq.astype(jnp.bfloat16)
    wo = wo.astype(jnp.bfloat16)
    k = jnp.swapaxes(K_ext, 1, 2).reshape(B * H_LOCAL, SQ, DH).astype(jnp.bfloat16)
    v = jnp.swapaxes(V_ext, 1, 2).reshape(B * H_LOCAL, SQ, DH).astype(jnp.bfloat16)

    out = pl.pallas_call(
        _body,
        out_shape=jax.ShapeDtypeStruct((B * SQ, D_MODEL), jnp.float32),
        in_specs=[pl.BlockSpec(memory_space=pltpu.VMEM)] * 5,
        out_specs=pl.BlockSpec(memory_space=pltpu.VMEM),
        scratch_shapes=[
            pltpu.VMEM((B * SQ, D_HEADS), jnp.bfloat16),
            pltpu.VMEM((3, B * SQ, D_MODEL), jnp.bfloat16),
            pltpu.VMEM((3, B * SQ, D_MODEL), jnp.bfloat16),
            pltpu.SemaphoreType.DMA((3, N_CHUNKS)),
            pltpu.SemaphoreType.DMA((3, N_CHUNKS)),
        ],
        compiler_params=pltpu.CompilerParams(collective_id=0),
    )(x2, wq, k, v, wo)
    return out.reshape(B, SQ, D_MODEL)


def _body(x_ref, wq_ref, k_ref, v_ref, wo_ref, out_ref, ctx_ref, send_ref,
          recv_ref, send_sems, recv_sems):
    my = lax.axis_index("i")

    barrier = pltpu.get_barrier_semaphore()
    for m in XOR_MASKS:
        pl.semaphore_signal(barrier, inc=1, device_id=(my ^ m,),
                            device_id_type=pl.DeviceIdType.MESH)
    pl.semaphore_wait(barrier, len(XOR_MASKS))

    qb = lax.broadcasted_iota(jnp.int32, (SQ, SQ), 0) // 64
    kb = lax.broadcasted_iota(jnp.int32, (SQ, SQ), 1) // 64
    mask = (qb == kb) | (kb == 0) | ((qb + kb) % 3 == 0)

    for b in range(B):
        xb = x_ref[b * SQ:(b + 1) * SQ, :]
        q_b = jnp.dot(xb, wq_ref[...], preferred_element_type=jnp.float32)
        q_b = q_b.astype(jnp.bfloat16)
        for h in range(H_LOCAL):
            qh = q_b[:, h * DH:(h + 1) * DH]
            kh = k_ref[b * H_LOCAL + h]
            s = lax.dot_general(qh, kh, (((1,), (1,)), ((), ())),
                                preferred_element_type=jnp.float32) * 0.125
            s = jnp.where(mask, s, -1e9)
            e = jnp.exp(s - jnp.max(s, axis=1, keepdims=True))
            w = (e / jnp.sum(e, axis=1, keepdims=True)).astype(jnp.bfloat16)
            ctx = jnp.dot(w, v_ref[b * H_LOCAL + h],
                          preferred_element_type=jnp.float32)
            ctx_ref[b * SQ:(b + 1) * SQ, h * DH:(h + 1) * DH] = (
                ctx.astype(jnp.bfloat16))

    out_ref[...] = jnp.dot(ctx_ref[...], wo_ref[...],
                           preferred_element_type=jnp.float32)

    def _rdma(r, c, m):
        sl = pl.ds(c * CHUNK, CHUNK)
        return pltpu.make_async_remote_copy(
            src_ref=send_ref.at[r, sl, :],
            dst_ref=recv_ref.at[r, sl, :],
            send_sem=send_sems.at[r, c],
            recv_sem=recv_sems.at[r, c],
            device_id=(my ^ m,),
            device_id_type=pl.DeviceIdType.MESH,
        )

    rdmas = {
        (r, c): _rdma(r, c, m)
        for r, m in enumerate(XOR_MASKS)
        for c in range(N_CHUNKS)
    }
    send_ref[0] = out_ref[...].astype(jnp.bfloat16)
    for c in range(N_CHUNKS):
        rdmas[0, c].start()
    for r in range(3):
        for c in range(N_CHUNKS):
            sl = slice(c * CHUNK, (c + 1) * CHUNK)
            rdmas[r, c].wait()
            acc = out_ref[sl, :] + recv_ref[r, sl, :].astype(jnp.float32)
            out_ref[sl, :] = acc
            if r < 2:
                send_ref[r + 1, sl, :] = acc.astype(jnp.bfloat16)
                rdmas[r + 1, c].start()

    @functools.partial(pl.run_scoped, exit_sem=pltpu.SemaphoreType.REGULAR)
    def _(exit_sem):
        for m in XOR_MASKS:
            pl.semaphore_signal(exit_sem, inc=1, device_id=(my ^ m,),
                                device_id_type=pl.DeviceIdType.MESH)
        pl.semaphore_wait(exit_sem, len(XOR_MASKS))


# device time: 26127 ns/iter; 1.9713x vs baseline; 1.0294x over previous
import functools

import jax
import jax.numpy as jnp
from jax import lax
from jax.experimental import pallas as pl
from jax.experimental.pallas import tpu as pltpu

N_DEV = 8
XOR_MASKS = (1, 3, 4)
N_CHUNKS = 8
CHUNK = 512 // N_CHUNKS
B, SQ, D_MODEL = 2, 256, 512
H_LOCAL, DH = 4, 64
D_HEADS = H_LOCAL * DH


def kernel(x, Wq, K_ext, V_ext, Wo):
    my = lax.axis_index("i")
    wq = lax.dynamic_slice_in_dim(Wq, my * D_HEADS, D_HEADS, axis=1)
    wo = lax.dynamic_slice_in_dim(Wo, my * D_HEADS, D_HEADS, axis=0)
    x2 = x.reshape(B * SQ, D_MODEL).astype(jnp.bfloat16)
    wq = wq.astype(jnp.bfloat16)
    wo = wo.astype(jnp.bfloat16)
    k = jnp.swapaxes(K_ext, 1, 2).reshape(B * H_LOCAL, SQ, DH).astype(jnp.bfloat16)
    v = jnp.swapaxes(V_ext, 1, 2).reshape(B * H_LOCAL, SQ, DH).astype(jnp.bfloat16)

    out = pl.pallas_call(
        _body,
        out_shape=jax.ShapeDtypeStruct((B * SQ, D_MODEL), jnp.float32),
        in_specs=[pl.BlockSpec(memory_space=pltpu.VMEM)] * 5,
        out_specs=pl.BlockSpec(memory_space=pltpu.VMEM),
        scratch_shapes=[
            pltpu.VMEM((B * SQ, D_HEADS), jnp.bfloat16),
            pltpu.VMEM((3, B * SQ, D_MODEL), jnp.bfloat16),
            pltpu.VMEM((3, B * SQ, D_MODEL), jnp.bfloat16),
            pltpu.SemaphoreType.DMA((3, N_CHUNKS)),
            pltpu.SemaphoreType.DMA((3, N_CHUNKS)),
        ],
        compiler_params=pltpu.CompilerParams(collective_id=0),
    )(x2, wq, k, v, wo)
    return out.reshape(B, SQ, D_MODEL)


def _body(x_ref, wq_ref, k_ref, v_ref, wo_ref, out_ref, ctx_ref, send_ref,
          recv_ref, send_sems, recv_sems):
    my = lax.axis_index("i")

    qb = lax.broadcasted_iota(jnp.int32, (SQ, SQ), 0) // 64
    kb = lax.broadcasted_iota(jnp.int32, (SQ, SQ), 1) // 64
    mask = (qb == kb) | (kb == 0) | ((qb + kb) % 3 == 0)

    def compute_batch(b):
        xb = x_ref[b * SQ:(b + 1) * SQ, :]
        q_b = jnp.dot(xb, wq_ref[...], preferred_element_type=jnp.float32)
        q_b = q_b.astype(jnp.bfloat16)
        for h in range(H_LOCAL):
            qh = q_b[:, h * DH:(h + 1) * DH]
            kh = k_ref[b * H_LOCAL + h]
            s = lax.dot_general(qh, kh, (((1,), (1,)), ((), ())),
                                preferred_element_type=jnp.float32) * 0.125
            s = jnp.where(mask, s, -1e9)
            e = jnp.exp(s - jnp.max(s, axis=1, keepdims=True))
            w = (e / jnp.sum(e, axis=1, keepdims=True)).astype(jnp.bfloat16)
            ctx = jnp.dot(w, v_ref[b * H_LOCAL + h],
                          preferred_element_type=jnp.float32)
            ctx_ref[b * SQ:(b + 1) * SQ, h * DH:(h + 1) * DH] = (
                ctx.astype(jnp.bfloat16))
        out_ref[b * SQ:(b + 1) * SQ, :] = jnp.dot(
            ctx_ref[b * SQ:(b + 1) * SQ, :], wo_ref[...],
            preferred_element_type=jnp.float32)

    def _rdma(r, c, m):
        sl = pl.ds(c * CHUNK, CHUNK)
        return pltpu.make_async_remote_copy(
            src_ref=send_ref.at[r, sl, :],
            dst_ref=recv_ref.at[r, sl, :],
            send_sem=send_sems.at[r, c],
            recv_sem=recv_sems.at[r, c],
            device_id=(my ^ m,),
            device_id_type=pl.DeviceIdType.MESH,
        )

    rdmas = {
        (r, c): _rdma(r, c, m)
        for r, m in enumerate(XOR_MASKS)
        for c in range(N_CHUNKS)
    }

    compute_batch(0)

    barrier = pltpu.get_barrier_semaphore()
    for m in XOR_MASKS:
        pl.semaphore_signal(barrier, inc=1, device_id=(my ^ m,),
                            device_id_type=pl.DeviceIdType.MESH)
    pl.semaphore_wait(barrier, len(XOR_MASKS))

    half = N_CHUNKS // 2
    send_ref[0, :SQ, :] = out_ref[:SQ, :].astype(jnp.bfloat16)
    for c in range(half):
        rdmas[0, c].start()
    compute_batch(1)
    send_ref[0, SQ:, :] = out_ref[SQ:, :].astype(jnp.bfloat16)
    for c in range(half, N_CHUNKS):
        rdmas[0, c].start()

    for r in range(3):
        for c in range(N_CHUNKS):
            sl = slice(c * CHUNK, (c + 1) * CHUNK)
            rdmas[r, c].wait()
            acc = out_ref[sl, :] + recv_ref[r, sl, :].astype(jnp.float32)
            out_ref[sl, :] = acc
            if r < 2:
                send_ref[r + 1, sl, :] = acc.astype(jnp.bfloat16)
                rdmas[r + 1, c].start()

    @functools.partial(pl.run_scoped, exit_sem=pltpu.SemaphoreType.REGULAR)
    def _(exit_sem):
        for m in XOR_MASKS:
            pl.semaphore_signal(exit_sem, inc=1, device_id=(my ^ m,),
                                device_id_type=pl.DeviceIdType.MESH)
        pl.semaphore_wait(exit_sem, len(XOR_MASKS))


# device time: 24811 ns/iter; 2.0758x vs baseline; 1.0530x over previous
import functools

import jax
import jax.numpy as jnp
from jax import lax
from jax.experimental import pallas as pl
from jax.experimental.pallas import tpu as pltpu

import os as _os
try:
    with open(_os.path.join(_os.path.dirname(_os.path.abspath(__file__)),
                            "ablate.txt")) as _f:
        _ABLATE = _f.read().strip()
except OSError:
    _ABLATE = ""

N_DEV = 8
XOR_MASKS = (1, 3, 4)
N_CHUNKS = 8
CHUNK = 512 // N_CHUNKS
B, SQ, D_MODEL = 2, 256, 512
H_LOCAL, DH = 4, 64
D_HEADS = H_LOCAL * DH


def kernel(x, Wq, K_ext, V_ext, Wo):
    my = lax.axis_index("i")
    wq = lax.dynamic_slice_in_dim(Wq, my * D_HEADS, D_HEADS, axis=1)
    wo = lax.dynamic_slice_in_dim(Wo, my * D_HEADS, D_HEADS, axis=0)
    x2 = x.reshape(B * SQ, D_MODEL).astype(jnp.bfloat16)
    wq = wq.astype(jnp.bfloat16)
    wo = wo.astype(jnp.bfloat16)
    k = jnp.swapaxes(K_ext, 1, 2).reshape(B * H_LOCAL, SQ, DH).astype(jnp.bfloat16)
    v = jnp.swapaxes(V_ext, 1, 2).reshape(B * H_LOCAL, SQ, DH).astype(jnp.bfloat16)

    out = pl.pallas_call(
        _body,
        out_shape=jax.ShapeDtypeStruct((B * SQ, D_MODEL), jnp.float32),
        in_specs=[pl.BlockSpec(memory_space=pltpu.VMEM)] * 5,
        out_specs=pl.BlockSpec(memory_space=pltpu.VMEM),
        scratch_shapes=[
            pltpu.VMEM((B * SQ, D_HEADS), jnp.bfloat16),
            pltpu.VMEM((3, B * SQ, D_MODEL), jnp.bfloat16),
            pltpu.VMEM((3, B * SQ, D_MODEL), jnp.bfloat16),
            pltpu.SemaphoreType.DMA((3, N_CHUNKS)),
            pltpu.SemaphoreType.DMA((3, N_CHUNKS)),
        ],
        compiler_params=(pltpu.CompilerParams() if _ABLATE == "compute_only"
                         else pltpu.CompilerParams(collective_id=0)),
    )(x2, wq, k, v, wo)
    return out.reshape(B, SQ, D_MODEL)


def _body(x_ref, wq_ref, k_ref, v_ref, wo_ref, out_ref, ctx_ref, send_ref,
          recv_ref, send_sems, recv_sems):
    my = lax.axis_index("i")

    qb = lax.broadcasted_iota(jnp.int32, (SQ, SQ), 0) // 64
    kb = lax.broadcasted_iota(jnp.int32, (SQ, SQ), 1) // 64
    mask = (qb == kb) | (kb == 0) | ((qb + kb) % 3 == 0)

    def compute_batch(b):
        xb = x_ref[b * SQ:(b + 1) * SQ, :]
        q_b = jnp.dot(xb, wq_ref[...], preferred_element_type=jnp.float32)
        q_b = q_b.astype(jnp.bfloat16)
        for h in range(H_LOCAL):
            qh = q_b[:, h * DH:(h + 1) * DH]
            kh = k_ref[b * H_LOCAL + h]
            s = lax.dot_general(qh, kh, (((1,), (1,)), ((), ())),
                                preferred_element_type=jnp.float32) * 0.125
            s = jnp.where(mask, s, -1e9)
            e = jnp.exp(s - jnp.max(s, axis=1, keepdims=True))
            w = (e / jnp.sum(e, axis=1, keepdims=True)).astype(jnp.bfloat16)
            ctx = jnp.dot(w, v_ref[b * H_LOCAL + h],
                          preferred_element_type=jnp.float32)
            ctx_ref[b * SQ:(b + 1) * SQ, h * DH:(h + 1) * DH] = (
                ctx.astype(jnp.bfloat16))
        out_ref[b * SQ:(b + 1) * SQ, :] = jnp.dot(
            ctx_ref[b * SQ:(b + 1) * SQ, :], wo_ref[...],
            preferred_element_type=jnp.float32)

    def _rdma(r, c, m):
        sl = pl.ds(c * CHUNK, CHUNK)
        return pltpu.make_async_remote_copy(
            src_ref=send_ref.at[r, sl, :],
            dst_ref=recv_ref.at[r, sl, :],
            send_sem=send_sems.at[r, c],
            recv_sem=recv_sems.at[r, c],
            device_id=(my ^ m,),
            device_id_type=pl.DeviceIdType.MESH,
        )

    rdmas = {
        (r, c): _rdma(r, c, m)
        for r, m in enumerate(XOR_MASKS)
        for c in range(N_CHUNKS)
    }

    if _ABLATE == "comm_only":
        out_ref[...] = jnp.zeros((B * SQ, D_MODEL), jnp.float32)
    else:
        compute_batch(0)
    if _ABLATE == "compute_only":
        compute_batch(1)
        return


    barrier = pltpu.get_barrier_semaphore()
    for m in XOR_MASKS:
        pl.semaphore_signal(barrier, inc=1, device_id=(my ^ m,),
                            device_id_type=pl.DeviceIdType.MESH)
    pl.semaphore_wait(barrier, len(XOR_MASKS))

    half = N_CHUNKS // 2
    send_ref[0, :SQ, :] = out_ref[:SQ, :].astype(jnp.bfloat16)
    for c in range(half):
        rdmas[0, c].start()
    if _ABLATE != "comm_only":
        compute_batch(1)
    send_ref[0, SQ:, :] = out_ref[SQ:, :].astype(jnp.bfloat16)
    for c in range(half, N_CHUNKS):
        rdmas[0, c].start()

    for r in range(3):
        for c in range(N_CHUNKS):
            sl = slice(c * CHUNK, (c + 1) * CHUNK)
            rdmas[r, c].wait()
            acc = out_ref[sl, :] + recv_ref[r, sl, :].astype(jnp.float32)
            out_ref[sl, :] = acc
            if r < 2:
                send_ref[r + 1, sl, :] = acc.astype(jnp.bfloat16)
                rdmas[r + 1, c].start()

    @functools.partial(pl.run_scoped, exit_sem=pltpu.SemaphoreType.REGULAR)
    def _(exit_sem):
        for m in XOR_MASKS:
            pl.semaphore_signal(exit_sem, inc=1, device_id=(my ^ m,),
                                device_id_type=pl.DeviceIdType.MESH)
        pl.semaphore_wait(exit_sem, len(XOR_MASKS))
